# Initial kernel scaffold; baseline (speedup 1.0000x reference)
import jax
import jax.numpy as jnp
from jax.experimental import pallas as pl


def kernel(x, edge_index, Wl1, Wr1, att1, b1, Wlin1, blin1, Wl2, Wr2, att2, b2, Wlin2, blin2):
    raise NotImplementedError("write your pallas kernel here")


# speedup vs baseline: 8.9339x; 8.9339x over previous
import functools

import jax
import jax.numpy as jnp
from jax import lax
from jax.experimental import pallas as pl
from jax.experimental.pallas import tpu as pltpu
from jax.experimental.pallas import tpu_sc as plsc

N = 10000
F = 128
E = 320000

NC = 2
NS = 16
NW = NC * NS
EPT = E // NW
B = 80
NB = EPT // B
NP = 10240
ROWS_PER_TILE = NP // NS
FC = F // 16


def _sc_edge_kernel_body(xl_hbm, xr_hbm, src_hbm, dst_hbm, att_hbm, zacc_hbm,
                         acc_out, den_out,
                         acc_sh, att_v, src_v, dst_v, xl_v, xr_v,
                         ptmp, a_v, den_l, sem0, sem1):
  cid = lax.axis_index("c")
  sid = lax.axis_index("s")
  wid = cid * NS + sid

  pltpu.sync_copy(att_hbm, att_v)
  zeros16 = jnp.zeros((16,), jnp.float32)

  def _zden(i, _):
    den_l[pl.ds(i * 16, 16)] = zeros16
    return 0

  lax.fori_loop(0, N // 16, _zden, 0)
  row0 = sid * ROWS_PER_TILE
  pltpu.sync_copy(zacc_hbm, acc_sh.at[pl.ds(row0, ROWS_PER_TILE)])
  plsc.subcore_barrier()

  lane = lax.iota(jnp.int32, 16)

  def _batch(b, _):
    base = wid * EPT + b * B
    pltpu.sync_copy(src_hbm.at[pl.ds(base, B)], src_v)
    pltpu.sync_copy(dst_hbm.at[pl.ds(base, B)], dst_v)
    cp0 = pltpu.async_copy(xl_hbm.at[src_v], xl_v, sem0)
    cp1 = pltpu.async_copy(xr_hbm.at[dst_v], xr_v, sem1)
    cp0.wait()
    cp1.wait()

    def _dot(i, _):
      p = jnp.zeros((16,), jnp.float32)
      for k in range(FC):
        z = xl_v[i, pl.ds(k * 16, 16)] + xr_v[i, pl.ds(k * 16, 16)]
        z = jnp.maximum(z, 0.2 * z)
        p = p + z * att_v[pl.ds(k * 16, 16)]
      plsc.store_scatter(ptmp, [lane * B + i], p)
      return 0

    lax.fori_loop(0, B, _dot, 0)

    for g in range(B // 16):
      s = ptmp[pl.ds(g * 16, 16)]
      for l in range(1, 16):
        s = s + ptmp[pl.ds(l * B + g * 16, 16)]
      a16 = jnp.exp(s)
      a_v[pl.ds(g * 16, 16)] = a16
      dst16 = dst_v[pl.ds(g * 16, 16)]
      plsc.addupdate_scatter(den_l, [dst16], a16)

    def _scale(i, _):
      ab = plsc.load_gather(a_v, [jnp.full((16,), i, jnp.int32)])
      for k in range(FC):
        xl_v[i, pl.ds(k * 16, 16)] = xl_v[i, pl.ds(k * 16, 16)] * ab
      return 0

    lax.fori_loop(0, B, _scale, 0)

    pltpu.sync_copy(xl_v, acc_sh.at[dst_v], add=True)
    return 0

  lax.fori_loop(0, NB, _batch, 0)

  plsc.subcore_barrier()
  pltpu.sync_copy(acc_sh.at[pl.ds(row0, ROWS_PER_TILE)],
                  acc_out.at[pl.ds(cid * NP + row0, ROWS_PER_TILE)])
  pltpu.sync_copy(den_l, den_out.at[pl.ds(wid * N, N)])


_sc_edge = functools.partial(
    pl.kernel,
    out_type=[
        jax.ShapeDtypeStruct((NC * NP, F), jnp.float32),
        jax.ShapeDtypeStruct((NW * N,), jnp.float32),
    ],
    mesh=plsc.VectorSubcoreMesh(core_axis_name="c", subcore_axis_name="s"),
    compiler_params=pltpu.CompilerParams(needs_layout_passes=False),
    scratch_types=[
        pltpu.VMEM_SHARED((NP, F), jnp.float32),
        pltpu.VMEM((F,), jnp.float32),
        pltpu.VMEM((B,), jnp.int32),
        pltpu.VMEM((B,), jnp.int32),
        pltpu.VMEM((B, F), jnp.float32),
        pltpu.VMEM((B, F), jnp.float32),
        pltpu.VMEM((16 * B,), jnp.float32),
        pltpu.VMEM((B,), jnp.float32),
        pltpu.VMEM((N,), jnp.float32),
        pltpu.SemaphoreType.DMA,
        pltpu.SemaphoreType.DMA,
    ],
)(_sc_edge_kernel_body)


def _mm_body(x_ref, w_ref, b_ref, o_ref):
  o_ref[...] = (
      jnp.dot(x_ref[...], w_ref[...], preferred_element_type=jnp.float32)
      + b_ref[...]
  )


def _mm384(x, wcat, bcat):
  blk = 1000
  return pl.pallas_call(
      _mm_body,
      grid=(N // blk,),
      in_specs=[
          pl.BlockSpec((blk, F), lambda i: (i, 0)),
          pl.BlockSpec((F, 3 * F), lambda i: (0, 0)),
          pl.BlockSpec((1, 3 * F), lambda i: (0, 0)),
      ],
      out_specs=pl.BlockSpec((blk, 3 * F), lambda i: (i, 0)),
      out_shape=jax.ShapeDtypeStruct((N, 3 * F), jnp.float32),
  )(x, wcat, bcat)


def _combine_mm_body(acc_ref, den_ref, y_ref, b_ref, w_ref, bcat_ref, o_ref):
  den = jnp.sum(den_ref[...], axis=1)
  h = acc_ref[0] + acc_ref[1]
  h = h / (den[:, None] + 1e-16) + b_ref[...] + y_ref[:, 2 * F:]
  h = jnp.maximum(h, 0.0)
  o_ref[...] = (
      jnp.dot(h, w_ref[...], preferred_element_type=jnp.float32)
      + bcat_ref[...]
  )


def _combine_mm(acc, den, y, b, wcat, bcat):
  blk = 1000
  return pl.pallas_call(
      _combine_mm_body,
      grid=(N // blk,),
      in_specs=[
          pl.BlockSpec((2, blk, F), lambda i: (0, i, 0)),
          pl.BlockSpec((blk, NW), lambda i: (i, 0)),
          pl.BlockSpec((blk, 3 * F), lambda i: (i, 0)),
          pl.BlockSpec((1, F), lambda i: (0, 0)),
          pl.BlockSpec((F, 3 * F), lambda i: (0, 0)),
          pl.BlockSpec((1, 3 * F), lambda i: (0, 0)),
      ],
      out_specs=pl.BlockSpec((blk, 3 * F), lambda i: (i, 0)),
      out_shape=jax.ShapeDtypeStruct((N, 3 * F), jnp.float32),
  )(acc, den, y, b, wcat, bcat)


def _final_body(acc_ref, den_ref, y_ref, b_ref, o_ref):
  den = jnp.sum(den_ref[...], axis=1)
  h = acc_ref[0] + acc_ref[1]
  o_ref[...] = h / (den[:, None] + 1e-16) + b_ref[...] + y_ref[:, 2 * F:]


def _final(acc, den, y, b):
  blk = 1000
  return pl.pallas_call(
      _final_body,
      grid=(N // blk,),
      in_specs=[
          pl.BlockSpec((2, blk, F), lambda i: (0, i, 0)),
          pl.BlockSpec((blk, NW), lambda i: (i, 0)),
          pl.BlockSpec((blk, 3 * F), lambda i: (i, 0)),
          pl.BlockSpec((1, F), lambda i: (0, 0)),
      ],
      out_specs=pl.BlockSpec((blk, F), lambda i: (i, 0)),
      out_shape=jax.ShapeDtypeStruct((N, F), jnp.float32),
  )(acc, den, y, b)


def kernel(x, edge_index, Wl1, Wr1, att1, b1, Wlin1, blin1,
           Wl2, Wr2, att2, b2, Wlin2, blin2):
  src = edge_index[0].astype(jnp.int32)
  dst = edge_index[1].astype(jnp.int32)
  zacc = jnp.zeros((ROWS_PER_TILE, F), jnp.float32)

  w1 = jnp.concatenate([Wl1, Wr1, Wlin1], axis=1)
  bc1 = jnp.concatenate(
      [jnp.zeros((2 * F,), jnp.float32), blin1])[None, :]
  y1 = _mm384(x, w1, bc1)

  acc1, den1 = _sc_edge(y1[:, :F], y1[:, F:2 * F], src, dst, att1, zacc)
  acc1 = acc1.reshape(NC, NP, F)[:, :N]
  den1 = den1.reshape(NW, N).T

  w2 = jnp.concatenate([Wl2, Wr2, Wlin2], axis=1)
  bc2 = jnp.concatenate(
      [jnp.zeros((2 * F,), jnp.float32), blin2])[None, :]
  y2 = _combine_mm(acc1, den1, y1, b1[None, :], w2, bc2)

  acc2, den2 = _sc_edge(y2[:, :F], y2[:, F:2 * F], src, dst, att2, zacc)
  acc2 = acc2.reshape(NC, NP, F)[:, :N]
  den2 = den2.reshape(NW, N).T

  return _final(acc2, den2, y2, b2[None, :])


# speedup vs baseline: 13.0256x; 1.4580x over previous
import functools

import jax
import jax.numpy as jnp
from jax import lax
from jax.experimental import pallas as pl
from jax.experimental.pallas import tpu as pltpu
from jax.experimental.pallas import tpu_sc as plsc

N = 10000
F = 128
E = 320000

NC = 2
NS = 16
NW = NC * NS
EPT = E // NW
B = 80
NB = EPT // B
NP = 10240
ROWS_PER_TILE = NP // NS
FC = F // 16


def _sc_edge_kernel_body(xl_hbm, xr_hbm, idx_hbm, att_hbm, zacc_hbm, zden_hbm,
                         acc_out, den_out,
                         acc_sh, den_sh, att_v,
                         iv0, iv1, xl_v0, xr_v0, xl_v1, xr_v1,
                         ptmp, a_v0, a_v1,
                         sgl0, sgr0, sgl1, sgr1, ss0, ss1, sd0, sd1):
  cid = lax.axis_index("c")
  sid = lax.axis_index("s")
  wid = cid * NS + sid

  pltpu.sync_copy(att_hbm, att_v)
  row0 = sid * ROWS_PER_TILE
  pltpu.sync_copy(zacc_hbm, acc_sh.at[pl.ds(row0, ROWS_PER_TILE)])
  pltpu.sync_copy(zden_hbm, den_sh.at[pl.ds(row0, ROWS_PER_TILE)])
  plsc.subcore_barrier()

  lane = lax.iota(jnp.int32, 16)
  bufs = (
      (iv0, xl_v0, xr_v0, a_v0, sgl0, sgr0, ss0, sd0),
      (iv1, xl_v1, xr_v1, a_v1, sgl1, sgr1, ss1, sd1),
  )

  def _issue_gather(b, buf):
    iv, xl_v, xr_v, _, sgl, sgr, _, _ = bufs[buf]
    pltpu.sync_copy(idx_hbm.at[wid * NB + b], iv)
    pltpu.async_copy(xl_hbm.at[iv.at[0]], xl_v, sgl)
    pltpu.async_copy(xr_hbm.at[iv.at[1]], xr_v, sgr)

  def _wait_gather(buf):
    iv, xl_v, xr_v, _, sgl, sgr, _, _ = bufs[buf]
    pltpu.make_async_copy(xl_hbm.at[iv.at[0]], xl_v, sgl).wait()
    pltpu.make_async_copy(xr_hbm.at[iv.at[1]], xr_v, sgr).wait()

  def _compute(buf):
    iv, xl_v, xr_v, a_v, _, _, ss, sd = bufs[buf]

    def _dot(i, _):
      p = jnp.zeros((16,), jnp.float32)
      for k in range(FC):
        z = xl_v[i, pl.ds(k * 16, 16)] + xr_v[i, pl.ds(k * 16, 16)]
        z = jnp.maximum(z, 0.2 * z)
        p = p + z * att_v[pl.ds(k * 16, 16)]
      plsc.store_scatter(ptmp, [lane * B + i], p)
      return 0

    lax.fori_loop(0, B, _dot, 0)

    for g in range(B // 16):
      s = ptmp[pl.ds(g * 16, 16)]
      for l in range(1, 16):
        s = s + ptmp[pl.ds(l * B + g * 16, 16)]
      a_v[pl.ds(g * 16, 16)] = jnp.exp(s)

    def _scale(i, _):
      ab = plsc.load_gather(a_v, [jnp.full((16,), i, jnp.int32)])
      for k in range(FC):
        xl_v[i, pl.ds(k * 16, 16)] = xl_v[i, pl.ds(k * 16, 16)] * ab
      return 0

    lax.fori_loop(0, B, _scale, 0)
    pltpu.async_copy(xl_v, acc_sh.at[iv.at[1]], ss, add=True)
    pltpu.async_copy(a_v, den_sh.at[iv.at[1]], sd, add=True)

  def _wait_scatter(buf):
    iv, xl_v, _, a_v, _, _, ss, sd = bufs[buf]
    pltpu.make_async_copy(xl_v, acc_sh.at[iv.at[1]], ss).wait()
    pltpu.make_async_copy(a_v, den_sh.at[iv.at[1]], sd).wait()

  _issue_gather(0, 0)

  def _pair(i, _):
    b1 = 2 * i + 1

    @pl.when(i > 0)
    def _():
      _wait_scatter(1)

    _issue_gather(b1, 1)
    _wait_gather(0)
    _compute(0)
    _wait_scatter(0)
    _issue_gather(b1 + 1, 0)
    _wait_gather(1)
    _compute(1)
    return 0

  lax.fori_loop(0, (NB - 1) // 2, _pair, 0)
  _wait_scatter(1)
  _wait_gather(0)
  _compute(0)
  _wait_scatter(0)

  plsc.subcore_barrier()
  pltpu.sync_copy(acc_sh.at[pl.ds(row0, ROWS_PER_TILE)],
                  acc_out.at[pl.ds(cid * NP + row0, ROWS_PER_TILE)])
  pltpu.sync_copy(den_sh.at[pl.ds(row0, ROWS_PER_TILE)],
                  den_out.at[pl.ds(cid * NP + row0, ROWS_PER_TILE)])


_sc_edge = functools.partial(
    pl.kernel,
    out_type=[
        jax.ShapeDtypeStruct((NC * NP, F), jnp.float32),
        jax.ShapeDtypeStruct((NC * NP,), jnp.float32),
    ],
    mesh=plsc.VectorSubcoreMesh(core_axis_name="c", subcore_axis_name="s"),
    compiler_params=pltpu.CompilerParams(needs_layout_passes=False),
    scratch_types=[
        pltpu.VMEM_SHARED((NP, F), jnp.float32),
        pltpu.VMEM_SHARED((NP,), jnp.float32),
        pltpu.VMEM((F,), jnp.float32),
        pltpu.VMEM((2, B), jnp.int32),
        pltpu.VMEM((2, B), jnp.int32),
        pltpu.VMEM((B, F), jnp.float32),
        pltpu.VMEM((B, F), jnp.float32),
        pltpu.VMEM((B, F), jnp.float32),
        pltpu.VMEM((B, F), jnp.float32),
        pltpu.VMEM((16 * B,), jnp.float32),
        pltpu.VMEM((B,), jnp.float32),
        pltpu.VMEM((B,), jnp.float32),
    ] + [pltpu.SemaphoreType.DMA] * 8,
)(_sc_edge_kernel_body)


def _mm_body(x_ref, w_ref, b_ref, xl_ref, xr_ref, lin_ref):
  y = (
      jnp.dot(x_ref[...], w_ref[...], preferred_element_type=jnp.float32)
      + b_ref[...]
  )
  xl_ref[...] = y[:, :F]
  xr_ref[...] = y[:, F:2 * F]
  lin_ref[...] = y[:, 2 * F:]


def _mm384(x, wcat, bcat):
  blk = 1000
  return pl.pallas_call(
      _mm_body,
      grid=(N // blk,),
      in_specs=[
          pl.BlockSpec((blk, F), lambda i: (i, 0)),
          pl.BlockSpec((F, 3 * F), lambda i: (0, 0)),
          pl.BlockSpec((1, 3 * F), lambda i: (0, 0)),
      ],
      out_specs=[pl.BlockSpec((blk, F), lambda i: (i, 0))] * 3,
      out_shape=[jax.ShapeDtypeStruct((N, F), jnp.float32)] * 3,
  )(x, wcat, bcat)


def _combine_mm_body(acc_ref, den_ref, lin_ref, b_ref, w_ref, bcat_ref,
                     xl_ref, xr_ref, lin2_ref):
  den = jnp.sum(den_ref[...], axis=1)
  h = acc_ref[0] + acc_ref[1]
  h = h / (den[:, None] + 1e-16) + b_ref[...] + lin_ref[...]
  h = jnp.maximum(h, 0.0)
  y = (
      jnp.dot(h, w_ref[...], preferred_element_type=jnp.float32)
      + bcat_ref[...]
  )
  xl_ref[...] = y[:, :F]
  xr_ref[...] = y[:, F:2 * F]
  lin2_ref[...] = y[:, 2 * F:]


def _combine_mm(acc, den, lin, b, wcat, bcat):
  blk = 1000
  return pl.pallas_call(
      _combine_mm_body,
      grid=(N // blk,),
      in_specs=[
          pl.BlockSpec((2, blk, F), lambda i: (0, i, 0)),
          pl.BlockSpec((blk, NC), lambda i: (i, 0)),
          pl.BlockSpec((blk, F), lambda i: (i, 0)),
          pl.BlockSpec((1, F), lambda i: (0, 0)),
          pl.BlockSpec((F, 3 * F), lambda i: (0, 0)),
          pl.BlockSpec((1, 3 * F), lambda i: (0, 0)),
      ],
      out_specs=[pl.BlockSpec((blk, F), lambda i: (i, 0))] * 3,
      out_shape=[jax.ShapeDtypeStruct((N, F), jnp.float32)] * 3,
  )(acc, den, lin, b, wcat, bcat)


def _final_body(acc_ref, den_ref, lin_ref, b_ref, o_ref):
  den = jnp.sum(den_ref[...], axis=1)
  h = acc_ref[0] + acc_ref[1]
  o_ref[...] = h / (den[:, None] + 1e-16) + b_ref[...] + lin_ref[...]


def _final(acc, den, lin, b):
  blk = 1000
  return pl.pallas_call(
      _final_body,
      grid=(N // blk,),
      in_specs=[
          pl.BlockSpec((2, blk, F), lambda i: (0, i, 0)),
          pl.BlockSpec((blk, NC), lambda i: (i, 0)),
          pl.BlockSpec((blk, F), lambda i: (i, 0)),
          pl.BlockSpec((1, F), lambda i: (0, 0)),
      ],
      out_specs=pl.BlockSpec((blk, F), lambda i: (i, 0)),
      out_shape=jax.ShapeDtypeStruct((N, F), jnp.float32),
  )(acc, den, lin, b)


def kernel(x, edge_index, Wl1, Wr1, att1, b1, Wlin1, blin1,
           Wl2, Wr2, att2, b2, Wlin2, blin2):
  src = edge_index[0].astype(jnp.int32).reshape(NW * NB, 1, B)
  dst = edge_index[1].astype(jnp.int32).reshape(NW * NB, 1, B)
  idx2 = jnp.concatenate([src, dst], axis=1)
  zacc = jnp.zeros((ROWS_PER_TILE, F), jnp.float32)
  zden = jnp.zeros((ROWS_PER_TILE,), jnp.float32)

  w1 = jnp.concatenate([Wl1, Wr1, Wlin1], axis=1)
  bc1 = jnp.concatenate(
      [jnp.zeros((2 * F,), jnp.float32), blin1])[None, :]
  xl1, xr1, lin1 = _mm384(x, w1, bc1)

  acc1, den1 = _sc_edge(xl1, xr1, idx2, att1, zacc, zden)
  acc1 = acc1.reshape(NC, NP, F)[:, :N]
  den1 = den1.reshape(NC, NP)[:, :N].T

  w2 = jnp.concatenate([Wl2, Wr2, Wlin2], axis=1)
  bc2 = jnp.concatenate(
      [jnp.zeros((2 * F,), jnp.float32), blin2])[None, :]
  xl2, xr2, lin2 = _combine_mm(acc1, den1, lin1, b1[None, :], w2, bc2)

  acc2, den2 = _sc_edge(xl2, xr2, idx2, att2, zacc, zden)
  acc2 = acc2.reshape(NC, NP, F)[:, :N]
  den2 = den2.reshape(NC, NP)[:, :N].T

  return _final(acc2, den2, lin2, b2[None, :])
